# baseline (speedup 1.0000x reference)
import functools

import jax
import jax.numpy as jnp
from jax import lax
from jax.experimental import pallas as pl
from jax.experimental.pallas import tpu as pltpu
from jax.experimental.pallas import tpu_sc as plsc

NN = 10000
NPAD = 10240
EE = 320000
FD = 128
NG = 64
GARB = 10000
NSC = 2
NSUB = 16
NTILES = NSC * NSUB
CPT = 80
NCH = NTILES * CPT
EPAD = NCH * 128
RPT = NPAD // NSUB
RATIO = 0.8
BLK = 1024
GRID = NPAD // BLK
NROW8 = NPAD // 8
NI = NPAD // 128
JT = 128
GT = 64
F32 = jnp.float32
I32 = jnp.int32


def _sc_mesh():
    return plsc.VectorSubcoreMesh(core_axis_name="c", subcore_axis_name="s")


def _wid():
    return lax.axis_index("c") * NSUB + lax.axis_index("s")


def _sc_deg_body(colt, degp, colv, onesv, zbuf, deg_sh):
    c = lax.axis_index("c")
    s = lax.axis_index("s")
    w = _wid()
    pltpu.sync_copy(colt.at[pl.ds(w * CPT, CPT)], colv)

    def zb(i, _):
        zbuf[pl.ds(i * 16, 16)] = jnp.zeros((16,), F32)
        return 0

    lax.fori_loop(0, RPT // 16, zb, 0)
    for u in range(8):
        onesv[pl.ds(u * 16, 16)] = jnp.ones((16,), F32)
    pltpu.sync_copy(zbuf, deg_sh.at[pl.ds(s * RPT, RPT)])
    plsc.subcore_barrier()

    def step(j, _):
        pltpu.sync_copy(onesv, deg_sh.at[colv.at[j]], add=True)
        return 0

    lax.fori_loop(0, CPT, step, 0)
    plsc.subcore_barrier()
    pltpu.sync_copy(deg_sh.at[pl.ds(s * RPT, RPT)], zbuf)
    pltpu.sync_copy(zbuf, degp.at[c, pl.ds(s * RPT, RPT)])


def _sc_deg_call(colt):
    return pl.kernel(
        _sc_deg_body,
        out_type=jax.ShapeDtypeStruct((NSC, NPAD), F32),
        mesh=_sc_mesh(),
        scratch_types=[
            pltpu.VMEM((CPT, 128), I32),
            pltpu.VMEM((128,), F32),
            pltpu.VMEM((RPT,), F32),
            pltpu.VMEM_SHARED((NPAD,), F32),
        ],
    )(colt)


NBUF = 8
GAHEAD = 4
ILEAD = 2
NIB = NBUF + ILEAD + 1
HF = FD // 2
TCPT = NCH // NSUB


def _sc_msg_body(hs2, idxt, accp, idxv, buf, acc_sh, isem, gsem, ssem):
    c = lax.axis_index("c")
    s = lax.axis_index("s")
    base = s * TCPT

    def zb(i, _):
        for u in range(HF // 16):
            buf[0, i, pl.ds(u * 16, 16)] = jnp.zeros((16,), F32)
        return 0

    lax.fori_loop(0, 128, zb, 0)

    def zc(m, _):
        pltpu.sync_copy(buf.at[0], acc_sh.at[pl.ds(s * RPT + m * 128, 128)])
        return 0

    lax.fori_loop(0, RPT // 128, zc, 0)
    plsc.subcore_barrier()

    def i_start(ch):
        sl = lax.rem(ch, NIB)
        pltpu.make_async_copy(idxt.at[c, base + ch], idxv.at[sl],
                              isem.at[sl]).start()

    def i_wait(ch):
        sl = lax.rem(ch, NIB)
        pltpu.make_async_copy(idxt.at[c, base + ch], idxv.at[sl],
                              isem.at[sl]).wait()

    def g_start(ch):
        sl = lax.rem(ch, NIB)
        b = lax.rem(ch, NBUF)
        pltpu.make_async_copy(hs2.at[idxv.at[sl, 0]], buf.at[b],
                              gsem.at[b]).start()

    def g_wait(ch):
        sl = lax.rem(ch, NIB)
        b = lax.rem(ch, NBUF)
        pltpu.make_async_copy(hs2.at[idxv.at[sl, 0]], buf.at[b],
                              gsem.at[b]).wait()

    def s_start(ch):
        sl = lax.rem(ch, NIB)
        b = lax.rem(ch, NBUF)
        pltpu.make_async_copy(buf.at[b], acc_sh.at[idxv.at[sl, 1]],
                              ssem.at[b]).start(add=True)

    def s_wait(b):
        pltpu.make_async_copy(buf.at[b], acc_sh.at[idxv.at[0, 1]],
                              ssem.at[b]).wait()

    for ch in range(GAHEAD + ILEAD):
        i_start(ch)
    for ch in range(GAHEAD):
        i_wait(ch)
        g_start(ch)

    def step(j, _):
        ni = j + GAHEAD + ILEAD
        ng = j + GAHEAD

        @pl.when(ni < TCPT)
        def _():
            i_start(ni)

        @pl.when(ng < TCPT)
        def _():
            @pl.when(j >= NBUF - GAHEAD)
            def _():
                s_wait(lax.rem(ng, NBUF))

            i_wait(ng)
            g_start(ng)

        g_wait(j)
        s_start(j)
        return 0

    lax.fori_loop(0, TCPT, step, 0)
    for ch in range(TCPT - NBUF, TCPT):
        s_wait(ch % NBUF)
    plsc.subcore_barrier()

    def ex(m, _):
        pltpu.sync_copy(acc_sh.at[pl.ds(s * RPT + m * 128, 128)], buf.at[0])
        pltpu.sync_copy(buf.at[0], accp.at[c, pl.ds(s * RPT + m * 128, 128)])
        return 0

    lax.fori_loop(0, RPT // 128, ex, 0)


def _sc_msg_call(hs2, idxt):
    return pl.kernel(
        _sc_msg_body,
        out_type=jax.ShapeDtypeStruct((NSC, NPAD, HF), F32),
        mesh=_sc_mesh(),
        scratch_types=[
            pltpu.VMEM((NIB, 2, 128), I32),
            pltpu.VMEM((NBUF, 128, HF), F32),
            pltpu.VMEM_SHARED((NPAD, HF), F32),
            pltpu.SemaphoreType.DMA((NIB,)),
            pltpu.SemaphoreType.DMA((NBUF,)),
            pltpu.SemaphoreType.DMA((NBUF,)),
        ],
        compiler_params=pltpu.CompilerParams(use_tc_tiling_on_sc=False),
    )(hs2, idxt)


def _sc_mask_body(mask, rowt, colt, degp, ceff, rowv, colv, maskv, emv, ceffv,
                  zbuf, deg_sh):
    c = lax.axis_index("c")
    s = lax.axis_index("s")
    w = _wid()
    pltpu.sync_copy(mask, maskv)
    pltpu.sync_copy(rowt.at[pl.ds(w * CPT, CPT)], rowv)
    pltpu.sync_copy(colt.at[pl.ds(w * CPT, CPT)], colv)

    def zb(i, _):
        zbuf[pl.ds(i * 16, 16)] = jnp.zeros((16,), F32)
        return 0

    lax.fori_loop(0, RPT // 16, zb, 0)
    pltpu.sync_copy(zbuf, deg_sh.at[pl.ds(s * RPT, RPT)])
    plsc.subcore_barrier()

    def step(j, _):
        for u in range(8):
            ri = rowv[j, pl.ds(u * 16, 16)]
            ci = colv[j, pl.ds(u * 16, 16)]
            mr = plsc.load_gather(maskv, [ri])
            mc = plsc.load_gather(maskv, [ci])
            em = mr * mc
            emv[pl.ds(u * 16, 16)] = em
            garb = GARB + u * 16 + lax.broadcasted_iota(I32, (16,), 0)
            ceffv[j, pl.ds(u * 16, 16)] = jnp.where(em > 0.0, ci, garb)
        pltpu.sync_copy(emv, deg_sh.at[colv.at[j]], add=True)
        return 0

    lax.fori_loop(0, CPT, step, 0)
    pltpu.sync_copy(ceffv, ceff.at[pl.ds(w * CPT, CPT)])
    plsc.subcore_barrier()
    pltpu.sync_copy(deg_sh.at[pl.ds(s * RPT, RPT)], zbuf)
    pltpu.sync_copy(zbuf, degp.at[c, pl.ds(s * RPT, RPT)])


def _sc_mask_call(mask, rowt, colt):
    return pl.kernel(
        _sc_mask_body,
        out_type=[
            jax.ShapeDtypeStruct((NSC, NPAD), F32),
            jax.ShapeDtypeStruct((NCH, 128), I32),
        ],
        mesh=_sc_mesh(),
        scratch_types=[
            pltpu.VMEM((CPT, 128), I32),
            pltpu.VMEM((CPT, 128), I32),
            pltpu.VMEM((NPAD,), F32),
            pltpu.VMEM((128,), F32),
            pltpu.VMEM((CPT, 128), I32),
            pltpu.VMEM((RPT,), F32),
            pltpu.VMEM_SHARED((NPAD,), F32),
        ],
        compiler_params=pltpu.CompilerParams(needs_layout_passes=False),
    )(mask, rowt, colt)


def _mm_body(x_ref, sc_ref, mk_ref, w_ref, d0_ref, d1_ref, nm_ref,
             hp_ref, hs_ref, dis_ref):
    xe = x_ref[...] * sc_ref[...] * mk_ref[...]
    hp = jnp.dot(xe, w_ref[...], preferred_element_type=F32)
    deg = d0_ref[...] + d1_ref[...] + nm_ref[...]
    dis = jnp.where(deg > 0.0, lax.rsqrt(deg), 0.0)
    hp_ref[...] = hp
    hs = hp * dis
    hs_ref[0] = hs[:, :HF]
    hs_ref[1] = hs[:, HF:]
    dis_ref[...] = dis


def _tc_mm(x, scv, mkv, w, d0, d1, nm):
    rblk = lambda i: (i, 0)
    return pl.pallas_call(
        _mm_body,
        grid=(GRID,),
        in_specs=[
            pl.BlockSpec((BLK, FD), rblk),
            pl.BlockSpec((BLK, 1), rblk),
            pl.BlockSpec((BLK, 1), rblk),
            pl.BlockSpec((FD, FD), lambda i: (0, 0)),
            pl.BlockSpec((BLK, 1), rblk),
            pl.BlockSpec((BLK, 1), rblk),
            pl.BlockSpec((BLK, 1), rblk),
        ],
        out_specs=[
            pl.BlockSpec((BLK, FD), rblk),
            pl.BlockSpec((NSC, BLK, HF), lambda i: (0, i, 0)),
            pl.BlockSpec((BLK, 1), rblk),
        ],
        out_shape=[
            jax.ShapeDtypeStruct((NPAD, FD), F32),
            jax.ShapeDtypeStruct((NSC, NPAD, HF), F32),
            jax.ShapeDtypeStruct((NPAD, 1), F32),
        ],
    )(x, scv, mkv, w, d0, d1, nm)


def _comb_body(acc_ref, hp_ref, dis_ref, nm_ref, b_ref, p_ref,
               h_ref, sc_ref, sr_ref):
    a = jnp.concatenate([acc_ref[0], acc_ref[1]], axis=1)
    dis = dis_ref[...]
    nm = nm_ref[...]
    h = jnp.maximum((a * dis + hp_ref[...] * (dis * dis) + b_ref[...]) * nm,
                    0.0)
    p = p_ref[...]
    pn = lax.rsqrt(jnp.sum(p * p))
    score = jnp.tanh(jnp.sum(h * p, axis=1, keepdims=True) * pn)
    h_ref[...] = h
    sc_ref[...] = score
    sr_ref[...] = jnp.where(nm > 0.0, score, -2.0)


def _tc_comb(accp, hp, dis, nm, b, p):
    rblk = lambda i: (i, 0)
    return pl.pallas_call(
        _comb_body,
        grid=(GRID,),
        in_specs=[
            pl.BlockSpec((NSC, BLK, HF), lambda i: (0, i, 0)),
            pl.BlockSpec((BLK, FD), rblk),
            pl.BlockSpec((BLK, 1), rblk),
            pl.BlockSpec((BLK, 1), rblk),
            pl.BlockSpec((1, FD), lambda i: (0, 0)),
            pl.BlockSpec((1, FD), lambda i: (0, 0)),
        ],
        out_specs=[
            pl.BlockSpec((BLK, FD), rblk),
            pl.BlockSpec((BLK, 1), rblk),
            pl.BlockSpec((BLK, 1), rblk),
        ],
        out_shape=[
            jax.ShapeDtypeStruct((NPAD, FD), F32),
            jax.ShapeDtypeStruct((NPAD, 1), F32),
            jax.ShapeDtypeStruct((NPAD, 1), F32),
        ],
    )(accp, hp, dis, nm, b, p)


def _rank_body(sr2d, b2d, srcol, bcol, jlo, jhi, out_ref):
    def outer(t, _):
        i_sr = sr2d[pl.ds(t, 1), :]
        i_b = b2d[pl.ds(t, 1), :]
        idx_i = t * 128 + lax.broadcasted_iota(I32, (1, 128), 1)

        def inner(jb, cnt):
            j_sr = srcol[pl.ds(jb * JT, JT), :]
            j_b = bcol[pl.ds(jb * JT, JT), :]
            idx_j = jb * JT + lax.broadcasted_iota(I32, (JT, 1), 0)
            gt = (j_sr > i_sr) | ((j_sr == i_sr) & (idx_j < idx_i))
            hit = (j_b == i_b) & gt
            return cnt + jnp.where(hit, 1.0, 0.0)

        cnt = lax.fori_loop(jlo[t], jhi[t], inner,
                            jnp.zeros((JT, 128), F32))
        out_ref[pl.ds(t, 1), :] = jnp.sum(cnt, axis=0, keepdims=True)
        return 0

    lax.fori_loop(0, NI, outer, 0)


def _tc_rank(sr2d, b2d, srcol, bcol, jlo, jhi):
    return pl.pallas_call(
        _rank_body,
        in_specs=[
            pl.BlockSpec((NI, 128), None),
            pl.BlockSpec((NI, 128), None),
            pl.BlockSpec((NPAD, 1), None),
            pl.BlockSpec((NPAD, 1), None),
            pl.BlockSpec(memory_space=pltpu.SMEM),
            pl.BlockSpec(memory_space=pltpu.SMEM),
        ],
        out_specs=pl.BlockSpec((NI, 128), None),
        out_shape=jax.ShapeDtypeStruct((NI, 128), F32),
    )(sr2d, b2d, srcol, bcol, jlo, jhi)


def _sel(rank_ref, alive_ref, oh_ref, ohT_ref, mask_ref):
    counts = jnp.dot(ohT_ref[...], alive_ref[...], preferred_element_type=F32)
    kv = jnp.ceil(RATIO * counts)
    knode = jnp.dot(oh_ref[...], kv, preferred_element_type=F32)
    mask_ref[...] = jnp.where(
        (rank_ref[...] < knode) & (alive_ref[...] > 0.0), 1.0, 0.0)


def _readout(h_ref, sc_ref, mk_ref, bcol_ref, ohT_ref, rs_ref, re_ref,
             x_ref, hm_ref):
    hm_ref[...] = h_ref[...] * sc_ref[...] * mk_ref[...]
    sums = jnp.dot(ohT_ref[...], hm_ref[...], preferred_element_type=F32)
    cnts = jnp.dot(ohT_ref[...], mk_ref[...], preferred_element_type=F32)
    x_ref[:, FD:] = sums / jnp.maximum(cnts, 1.0)

    for g in range(NG):
        def inner(t, acc):
            rows = hm_ref[pl.ds(t * GT, GT), :]
            bt = bcol_ref[pl.ds(t * GT, GT), :]
            mr = mk_ref[pl.ds(t * GT, GT), :]
            val = jnp.where((mr > 0.0) & (bt == g), rows, -1e30)
            return jnp.maximum(acc, jnp.max(val, axis=0, keepdims=True))

        acc = lax.fori_loop(rs_ref[g], re_ref[g], inner,
                            jnp.full((1, FD), -1e30, F32))
        x_ref[g:g + 1, 0:FD] = jnp.where(acc < -1e29, 0.0, acc)


def _readout_body(h_ref, sc_ref, rank_ref, alive_ref, oh_ref, bcol_ref,
                  ohT_ref, rs_ref, re_ref, x_ref, mask_ref, hm_ref):
    _sel(rank_ref, alive_ref, oh_ref, ohT_ref, mask_ref)
    _readout(h_ref, sc_ref, mask_ref, bcol_ref, ohT_ref, rs_ref, re_ref,
             x_ref, hm_ref)


def _tc_readout(h, sc, rank, alive, oh, bcol, ohT, rs, re):
    return pl.pallas_call(
        _readout_body,
        in_specs=[
            pl.BlockSpec((NPAD, FD), None),
            pl.BlockSpec((NPAD, 1), None),
            pl.BlockSpec((NPAD, 1), None),
            pl.BlockSpec((NPAD, 1), None),
            pl.BlockSpec((NPAD, NG), None),
            pl.BlockSpec((NPAD, 1), None),
            pl.BlockSpec((NG, NPAD), None),
            pl.BlockSpec(memory_space=pltpu.SMEM),
            pl.BlockSpec(memory_space=pltpu.SMEM),
        ],
        out_shape=[jax.ShapeDtypeStruct((NG, 2 * FD), F32),
                   jax.ShapeDtypeStruct((NPAD, 1), F32)],
        scratch_shapes=[pltpu.VMEM((NPAD, FD), F32)],
    )(h, sc, rank, alive, oh, bcol, ohT, rs, re)


def _final_body(h_ref, sc_ref, rank_ref, alive_ref, oh_ref, bcol_ref,
                ohT_ref, rs_ref, re_ref,
                x1_ref, l1w_ref, l1b_ref, l2w_ref, l2b_ref,
                out_ref, mk_ref, hm_ref, x2_ref):
    _sel(rank_ref, alive_ref, oh_ref, ohT_ref, mk_ref)
    _readout(h_ref, sc_ref, mk_ref, bcol_ref, ohT_ref, rs_ref, re_ref,
             x2_ref, hm_ref)
    z = x1_ref[...] + x2_ref[...]
    z = jnp.maximum(
        jnp.dot(z, l1w_ref[...], preferred_element_type=F32) + l1b_ref[...],
        0.0)
    out_ref[...] = (jnp.dot(z, l2w_ref[...], preferred_element_type=F32)
                    + l2b_ref[...])


def _tc_final(h, sc, rank, alive, oh, bcol, ohT, rs, re, x1,
              l1w, l1b, l2w, l2b):
    return pl.pallas_call(
        _final_body,
        in_specs=[
            pl.BlockSpec((NPAD, FD), None),
            pl.BlockSpec((NPAD, 1), None),
            pl.BlockSpec((NPAD, 1), None),
            pl.BlockSpec((NPAD, 1), None),
            pl.BlockSpec((NPAD, NG), None),
            pl.BlockSpec((NPAD, 1), None),
            pl.BlockSpec((NG, NPAD), None),
            pl.BlockSpec(memory_space=pltpu.SMEM),
            pl.BlockSpec(memory_space=pltpu.SMEM),
            pl.BlockSpec((NG, 2 * FD), None),
            pl.BlockSpec((2 * FD, FD), None),
            pl.BlockSpec((1, FD), None),
            pl.BlockSpec((FD, NG), None),
            pl.BlockSpec((1, NG), None),
        ],
        out_shape=jax.ShapeDtypeStruct((NG, NG), F32),
        scratch_shapes=[pltpu.VMEM((NPAD, 1), F32),
                        pltpu.VMEM((NPAD, FD), F32),
                        pltpu.VMEM((NG, 2 * FD), F32)],
    )(h, sc, rank, alive, oh, bcol, ohT, rs, re, x1, l1w, l1b, l2w, l2b)


def kernel(x, edge_index, edge_attr, batch, W1, b1, p1, W2, b2, p2,
           lin1_W, lin1_b, lin2_W, lin2_b):
    del edge_attr
    xpad = jnp.pad(x, ((0, NPAD - NN), (0, 0)))
    row = edge_index[0].astype(I32)
    col = edge_index[1].astype(I32)
    rowt = jnp.concatenate(
        [row, jnp.zeros((EPAD - EE,), I32)]).reshape(NCH, 128)
    rowt2 = jnp.stack([rowt, rowt + NPAD])
    colt = jnp.concatenate(
        [col, GARB + jnp.arange(EPAD - EE, dtype=I32) % (NPAD - NN)]
    ).reshape(NCH, 128)
    bpad = jnp.concatenate(
        [batch.astype(I32), jnp.full((NPAD - NN,), NG - 1, I32)])
    bcol = bpad.reshape(NPAD, 1)
    b2d = bpad.reshape(NI, 128)
    valid = (jnp.arange(NPAD) < NN).astype(F32)
    vcol = valid.reshape(NPAD, 1)
    v2d = valid.reshape(NI, 128)
    oh = (bpad[:, None] == jnp.arange(NG)[None, :]).astype(F32) \
        * valid[:, None]
    ohT = oh.T
    starts = jnp.searchsorted(bpad, jnp.arange(NG)).astype(I32)
    ends = jnp.searchsorted(bpad, jnp.arange(NG), side="right").astype(I32)
    blo = b2d[:, 0]
    bhi = b2d[:, -1]
    jlo = (starts[blo] // JT).astype(I32)
    jhi = ((ends[bhi] + JT - 1) // JT).astype(I32)
    rs = (starts // GT).astype(I32)
    re = ((ends + GT - 1) // GT).astype(I32)
    ones_col = jnp.ones((NPAD, 1), F32)
    b1r = b1.reshape(1, FD)
    p1r = p1.reshape(1, FD)
    b2r = b2.reshape(1, FD)
    p2r = p2.reshape(1, FD)

    degp1 = _sc_deg_call(colt)
    d10 = degp1[0].reshape(NPAD, 1)
    d11 = degp1[1].reshape(NPAD, 1)
    h1p, h1s, dis1 = _tc_mm(xpad, ones_col, ones_col, W1, d10, d11, vcol)
    idxt1 = jnp.stack(
        [rowt2, jnp.broadcast_to(colt[None], (NSC, NCH, 128))], axis=2)
    accp1 = _sc_msg_call(h1s.reshape(NSC * NPAD, HF), idxt1)
    h1, score1, srank1 = _tc_comb(accp1, h1p, dis1, vcol, b1r, p1r)
    rank1 = _tc_rank(srank1.reshape(NI, 128), b2d, srank1, bcol, jlo, jhi)
    x1, mask1 = _tc_readout(h1, score1, rank1.reshape(NPAD, 1), vcol, oh,
                            bcol, ohT, rs, re)
    degp2, ceff = _sc_mask_call(mask1.reshape(NPAD), rowt, colt)
    d20 = degp2[0].reshape(NPAD, 1)
    d21 = degp2[1].reshape(NPAD, 1)
    h2p, h2s, dis2 = _tc_mm(h1, score1, mask1, W2, d20, d21, mask1)
    idxt2 = jnp.stack(
        [rowt2, jnp.broadcast_to(ceff[None], (NSC, NCH, 128))], axis=2)
    accp2 = _sc_msg_call(h2s.reshape(NSC * NPAD, HF), idxt2)
    h2, score2, srank2 = _tc_comb(accp2, h2p, dis2, mask1, b2r, p2r)
    m1_2d = mask1.reshape(NI, 128)
    rank2 = _tc_rank(srank2.reshape(NI, 128), b2d, srank2, bcol, jlo, jhi)
    return _tc_final(h2, score2, rank2.reshape(NPAD, 1), mask1, oh, bcol,
                     ohT, rs, re, x1, lin1_W, lin1_b.reshape(1, FD), lin2_W,
                     lin2_b.reshape(1, NG))


# speedup vs baseline: 1.0331x; 1.0331x over previous
import functools

import jax
import jax.numpy as jnp
from jax import lax
from jax.experimental import pallas as pl
from jax.experimental.pallas import tpu as pltpu
from jax.experimental.pallas import tpu_sc as plsc

NN = 10000
NPAD = 10240
EE = 320000
FD = 128
NG = 64
GARB = 10000
NSC = 2
NSUB = 16
NTILES = NSC * NSUB
CPT = 80
NCH = NTILES * CPT
EPAD = NCH * 128
RPT = NPAD // NSUB
RATIO = 0.8
BLK = 1024
GRID = NPAD // BLK
NROW8 = NPAD // 8
NI = NPAD // 128
JT = 128
GT = 64
F32 = jnp.float32
I32 = jnp.int32


def _sc_mesh():
    return plsc.VectorSubcoreMesh(core_axis_name="c", subcore_axis_name="s")


def _wid():
    return lax.axis_index("c") * NSUB + lax.axis_index("s")


def _sc_deg_body(colt, degp, colv, onesv, zbuf, deg_sh):
    c = lax.axis_index("c")
    s = lax.axis_index("s")
    w = _wid()
    pltpu.sync_copy(colt.at[pl.ds(w * CPT, CPT)], colv)

    def zb(i, _):
        zbuf[pl.ds(i * 16, 16)] = jnp.zeros((16,), F32)
        return 0

    lax.fori_loop(0, RPT // 16, zb, 0)
    for u in range(8):
        onesv[pl.ds(u * 16, 16)] = jnp.ones((16,), F32)
    pltpu.sync_copy(zbuf, deg_sh.at[pl.ds(s * RPT, RPT)])
    plsc.subcore_barrier()

    def step(j, _):
        pltpu.sync_copy(onesv, deg_sh.at[colv.at[j]], add=True)
        return 0

    lax.fori_loop(0, CPT, step, 0)
    plsc.subcore_barrier()
    pltpu.sync_copy(deg_sh.at[pl.ds(s * RPT, RPT)], zbuf)
    pltpu.sync_copy(zbuf, degp.at[c, pl.ds(s * RPT, RPT)])


def _sc_deg_call(colt):
    return pl.kernel(
        _sc_deg_body,
        out_type=jax.ShapeDtypeStruct((NSC, NPAD), F32),
        mesh=_sc_mesh(),
        scratch_types=[
            pltpu.VMEM((CPT, 128), I32),
            pltpu.VMEM((128,), F32),
            pltpu.VMEM((RPT,), F32),
            pltpu.VMEM_SHARED((NPAD,), F32),
        ],
    )(colt)


NBUF = 8
GAHEAD = 4
ILEAD = 2
NIB = NBUF + ILEAD + 1
HF = FD // 2
TCPT = NCH // NSUB


def _sc_msg_body(hs2, idxt, accp, idxv, buf, acc_sh, isem, gsem, ssem):
    c = lax.axis_index("c")
    s = lax.axis_index("s")
    base = s * TCPT

    def zb(i, _):
        for u in range(HF // 16):
            buf[0, i, pl.ds(u * 16, 16)] = jnp.zeros((16,), F32)
        return 0

    lax.fori_loop(0, 128, zb, 0)

    def zc(m, _):
        pltpu.sync_copy(buf.at[0], acc_sh.at[pl.ds(s * RPT + m * 128, 128)])
        return 0

    lax.fori_loop(0, RPT // 128, zc, 0)
    plsc.subcore_barrier()

    def i_start(ch):
        sl = lax.rem(ch, NIB)
        pltpu.make_async_copy(idxt.at[c, base + ch], idxv.at[sl],
                              isem.at[sl]).start()

    def i_wait(ch):
        sl = lax.rem(ch, NIB)
        pltpu.make_async_copy(idxt.at[c, base + ch], idxv.at[sl],
                              isem.at[sl]).wait()

    def g_start(ch):
        sl = lax.rem(ch, NIB)
        b = lax.rem(ch, NBUF)
        pltpu.make_async_copy(hs2.at[idxv.at[sl, 0]], buf.at[b],
                              gsem.at[b]).start()

    def g_wait(ch):
        sl = lax.rem(ch, NIB)
        b = lax.rem(ch, NBUF)
        pltpu.make_async_copy(hs2.at[idxv.at[sl, 0]], buf.at[b],
                              gsem.at[b]).wait()

    def s_start(ch):
        sl = lax.rem(ch, NIB)
        b = lax.rem(ch, NBUF)
        pltpu.make_async_copy(buf.at[b], acc_sh.at[idxv.at[sl, 1]],
                              ssem.at[b]).start(add=True)

    def s_wait(b):
        pltpu.make_async_copy(buf.at[b], acc_sh.at[idxv.at[0, 1]],
                              ssem.at[b]).wait()

    for ch in range(GAHEAD + ILEAD):
        i_start(ch)
    for ch in range(GAHEAD):
        i_wait(ch)
        g_start(ch)

    def step(j, _):
        ni = j + GAHEAD + ILEAD
        ng = j + GAHEAD

        @pl.when(ni < TCPT)
        def _():
            i_start(ni)

        @pl.when(ng < TCPT)
        def _():
            @pl.when(j >= NBUF - GAHEAD)
            def _():
                s_wait(lax.rem(ng, NBUF))

            i_wait(ng)
            g_start(ng)

        g_wait(j)
        s_start(j)
        return 0

    lax.fori_loop(0, TCPT, step, 0)
    for ch in range(TCPT - NBUF, TCPT):
        s_wait(ch % NBUF)
    plsc.subcore_barrier()

    def ex(m, _):
        pltpu.sync_copy(acc_sh.at[pl.ds(s * RPT + m * 128, 128)], buf.at[0])
        pltpu.sync_copy(buf.at[0], accp.at[c, pl.ds(s * RPT + m * 128, 128)])
        return 0

    lax.fori_loop(0, RPT // 128, ex, 0)


def _sc_msg_call(hs2, idxt):
    return pl.kernel(
        _sc_msg_body,
        out_type=jax.ShapeDtypeStruct((NSC, NPAD, HF), F32),
        mesh=_sc_mesh(),
        scratch_types=[
            pltpu.VMEM((NIB, 2, 128), I32),
            pltpu.VMEM((NBUF, 128, HF), F32),
            pltpu.VMEM_SHARED((NPAD, HF), F32),
            pltpu.SemaphoreType.DMA((NIB,)),
            pltpu.SemaphoreType.DMA((NBUF,)),
            pltpu.SemaphoreType.DMA((NBUF,)),
        ],
        compiler_params=pltpu.CompilerParams(use_tc_tiling_on_sc=False),
    )(hs2, idxt)


def _sc_mask_body(mask, rowt, colt, degp, ceff, rowv, colv, maskv, emv, ceffv,
                  zbuf, deg_sh):
    c = lax.axis_index("c")
    s = lax.axis_index("s")
    w = _wid()
    pltpu.sync_copy(mask, maskv)
    pltpu.sync_copy(rowt.at[pl.ds(w * CPT, CPT)], rowv)
    pltpu.sync_copy(colt.at[pl.ds(w * CPT, CPT)], colv)

    def zb(i, _):
        zbuf[pl.ds(i * 16, 16)] = jnp.zeros((16,), F32)
        return 0

    lax.fori_loop(0, RPT // 16, zb, 0)
    pltpu.sync_copy(zbuf, deg_sh.at[pl.ds(s * RPT, RPT)])
    plsc.subcore_barrier()

    def step(j, _):
        for u in range(8):
            ri = rowv[j, pl.ds(u * 16, 16)]
            ci = colv[j, pl.ds(u * 16, 16)]
            mr = plsc.load_gather(maskv, [ri])
            mc = plsc.load_gather(maskv, [ci])
            em = mr * mc
            emv[pl.ds(u * 16, 16)] = em
            garb = GARB + u * 16 + lax.broadcasted_iota(I32, (16,), 0)
            ceffv[j, pl.ds(u * 16, 16)] = jnp.where(em > 0.0, ci, garb)
        pltpu.sync_copy(emv, deg_sh.at[colv.at[j]], add=True)
        return 0

    lax.fori_loop(0, CPT, step, 0)
    pltpu.sync_copy(ceffv, ceff.at[pl.ds(w * CPT, CPT)])
    plsc.subcore_barrier()
    pltpu.sync_copy(deg_sh.at[pl.ds(s * RPT, RPT)], zbuf)
    pltpu.sync_copy(zbuf, degp.at[c, pl.ds(s * RPT, RPT)])


def _sc_mask_call(mask, rowt, colt):
    return pl.kernel(
        _sc_mask_body,
        out_type=[
            jax.ShapeDtypeStruct((NSC, NPAD), F32),
            jax.ShapeDtypeStruct((NCH, 128), I32),
        ],
        mesh=_sc_mesh(),
        scratch_types=[
            pltpu.VMEM((CPT, 128), I32),
            pltpu.VMEM((CPT, 128), I32),
            pltpu.VMEM((NPAD,), F32),
            pltpu.VMEM((128,), F32),
            pltpu.VMEM((CPT, 128), I32),
            pltpu.VMEM((RPT,), F32),
            pltpu.VMEM_SHARED((NPAD,), F32),
        ],
        compiler_params=pltpu.CompilerParams(needs_layout_passes=False),
    )(mask, rowt, colt)


def _mm_body(x_ref, sc_ref, mk_ref, w_ref, d0_ref, d1_ref, nm_ref,
             hp_ref, hs_ref, dis_ref):
    xe = x_ref[...] * sc_ref[...] * mk_ref[...]
    hp = jnp.dot(xe, w_ref[...], preferred_element_type=F32)
    deg = d0_ref[...] + d1_ref[...] + nm_ref[...]
    dis = jnp.where(deg > 0.0, lax.rsqrt(deg), 0.0)
    hp_ref[...] = hp
    hs = hp * dis
    hs_ref[0] = hs[:, :HF]
    hs_ref[1] = hs[:, HF:]
    dis_ref[...] = dis


def _tc_mm(x, scv, mkv, w, d0, d1, nm):
    rblk = lambda i: (i, 0)
    return pl.pallas_call(
        _mm_body,
        grid=(GRID,),
        in_specs=[
            pl.BlockSpec((BLK, FD), rblk),
            pl.BlockSpec((BLK, 1), rblk),
            pl.BlockSpec((BLK, 1), rblk),
            pl.BlockSpec((FD, FD), lambda i: (0, 0)),
            pl.BlockSpec((BLK, 1), rblk),
            pl.BlockSpec((BLK, 1), rblk),
            pl.BlockSpec((BLK, 1), rblk),
        ],
        out_specs=[
            pl.BlockSpec((BLK, FD), rblk),
            pl.BlockSpec((NSC, BLK, HF), lambda i: (0, i, 0)),
            pl.BlockSpec((BLK, 1), rblk),
        ],
        out_shape=[
            jax.ShapeDtypeStruct((NPAD, FD), F32),
            jax.ShapeDtypeStruct((NSC, NPAD, HF), F32),
            jax.ShapeDtypeStruct((NPAD, 1), F32),
        ],
    )(x, scv, mkv, w, d0, d1, nm)


def _comb_body(acc_ref, hp_ref, dis_ref, nm_ref, b_ref, p_ref,
               h_ref, sc_ref, sr_ref):
    a = jnp.concatenate([acc_ref[0], acc_ref[1]], axis=1)
    dis = dis_ref[...]
    nm = nm_ref[...]
    h = jnp.maximum((a * dis + hp_ref[...] * (dis * dis) + b_ref[...]) * nm,
                    0.0)
    p = p_ref[...]
    pn = lax.rsqrt(jnp.sum(p * p))
    score = jnp.tanh(jnp.sum(h * p, axis=1, keepdims=True) * pn)
    h_ref[...] = h
    sc_ref[...] = score
    sr_ref[...] = jnp.where(nm > 0.0, score, -2.0)


def _tc_comb(accp, hp, dis, nm, b, p):
    rblk = lambda i: (i, 0)
    return pl.pallas_call(
        _comb_body,
        grid=(GRID,),
        in_specs=[
            pl.BlockSpec((NSC, BLK, HF), lambda i: (0, i, 0)),
            pl.BlockSpec((BLK, FD), rblk),
            pl.BlockSpec((BLK, 1), rblk),
            pl.BlockSpec((BLK, 1), rblk),
            pl.BlockSpec((1, FD), lambda i: (0, 0)),
            pl.BlockSpec((1, FD), lambda i: (0, 0)),
        ],
        out_specs=[
            pl.BlockSpec((BLK, FD), rblk),
            pl.BlockSpec((BLK, 1), rblk),
            pl.BlockSpec((BLK, 1), rblk),
        ],
        out_shape=[
            jax.ShapeDtypeStruct((NPAD, FD), F32),
            jax.ShapeDtypeStruct((NPAD, 1), F32),
            jax.ShapeDtypeStruct((NPAD, 1), F32),
        ],
    )(accp, hp, dis, nm, b, p)


def _rank_body(sr2d, b2d, srcol, bcol, jlo, jhi, out_ref):
    def outer(t, _):
        i_sr = sr2d[pl.ds(t, 1), :]
        i_b = b2d[pl.ds(t, 1), :]
        idx_i = t * 128 + lax.broadcasted_iota(I32, (1, 128), 1)

        def inner(jb, cnt):
            j_sr = srcol[pl.ds(jb * JT, JT), :]
            j_b = bcol[pl.ds(jb * JT, JT), :]
            idx_j = jb * JT + lax.broadcasted_iota(I32, (JT, 1), 0)
            gt = (j_sr > i_sr) | ((j_sr == i_sr) & (idx_j < idx_i))
            hit = (j_b == i_b) & gt
            return cnt + jnp.where(hit, 1.0, 0.0)

        cnt = lax.fori_loop(jlo[t], jhi[t], inner,
                            jnp.zeros((JT, 128), F32))
        out_ref[pl.ds(t, 1), :] = jnp.sum(cnt, axis=0, keepdims=True)
        return 0

    lax.fori_loop(0, NI, outer, 0)


def _tc_rank(sr2d, b2d, srcol, bcol, jlo, jhi):
    return pl.pallas_call(
        _rank_body,
        in_specs=[
            pl.BlockSpec((NI, 128), None),
            pl.BlockSpec((NI, 128), None),
            pl.BlockSpec((NPAD, 1), None),
            pl.BlockSpec((NPAD, 1), None),
            pl.BlockSpec(memory_space=pltpu.SMEM),
            pl.BlockSpec(memory_space=pltpu.SMEM),
        ],
        out_specs=pl.BlockSpec((NI, 128), None),
        out_shape=jax.ShapeDtypeStruct((NI, 128), F32),
    )(sr2d, b2d, srcol, bcol, jlo, jhi)


def _sel_body(rank_ref, alive_ref, oh_ref, ohT_ref, mask_ref):
    counts = jnp.dot(ohT_ref[...], alive_ref[...], preferred_element_type=F32)
    kv = jnp.ceil(RATIO * counts)
    knode = jnp.dot(oh_ref[...], kv, preferred_element_type=F32)
    mask_ref[...] = jnp.where(
        (rank_ref[...] < knode) & (alive_ref[...] > 0.0), 1.0, 0.0)


def _tc_sel(rank, alive, oh, ohT):
    return pl.pallas_call(
        _sel_body,
        out_shape=jax.ShapeDtypeStruct((NPAD, 1), F32),
    )(rank, alive, oh, ohT)


def _readout(h_ref, sc_ref, mk_ref, bcol_ref, ohT_ref, rs_ref, re_ref,
             x_ref, hm_ref):
    hm_ref[...] = h_ref[...] * sc_ref[...] * mk_ref[...]
    sums = jnp.dot(ohT_ref[...], hm_ref[...], preferred_element_type=F32)
    cnts = jnp.dot(ohT_ref[...], mk_ref[...], preferred_element_type=F32)
    x_ref[:, FD:] = sums / jnp.maximum(cnts, 1.0)

    for g in range(NG):
        def inner(t, acc):
            rows = hm_ref[pl.ds(t * GT, GT), :]
            bt = bcol_ref[pl.ds(t * GT, GT), :]
            mr = mk_ref[pl.ds(t * GT, GT), :]
            val = jnp.where((mr > 0.0) & (bt == g), rows, -1e30)
            return jnp.maximum(acc, jnp.max(val, axis=0, keepdims=True))

        acc = lax.fori_loop(rs_ref[g], re_ref[g], inner,
                            jnp.full((1, FD), -1e30, F32))
        x_ref[g:g + 1, 0:FD] = jnp.where(acc < -1e29, 0.0, acc)


def _readout_body(h_ref, sc_ref, mk_ref, bcol_ref, ohT_ref, rs_ref, re_ref,
                  x_ref, hm_ref):
    _readout(h_ref, sc_ref, mk_ref, bcol_ref, ohT_ref, rs_ref, re_ref,
             x_ref, hm_ref)


def _tc_readout(h, sc, mk, bcol, ohT, rs, re):
    return pl.pallas_call(
        _readout_body,
        in_specs=[
            pl.BlockSpec((NPAD, FD), None),
            pl.BlockSpec((NPAD, 1), None),
            pl.BlockSpec((NPAD, 1), None),
            pl.BlockSpec((NPAD, 1), None),
            pl.BlockSpec((NG, NPAD), None),
            pl.BlockSpec(memory_space=pltpu.SMEM),
            pl.BlockSpec(memory_space=pltpu.SMEM),
        ],
        out_shape=jax.ShapeDtypeStruct((NG, 2 * FD), F32),
        scratch_shapes=[pltpu.VMEM((NPAD, FD), F32)],
    )(h, sc, mk, bcol, ohT, rs, re)


def _final_body(h_ref, sc_ref, mk_ref, bcol_ref, ohT_ref, rs_ref, re_ref,
                x1_ref, l1w_ref, l1b_ref, l2w_ref, l2b_ref,
                out_ref, hm_ref, x2_ref):
    _readout(h_ref, sc_ref, mk_ref, bcol_ref, ohT_ref, rs_ref, re_ref,
             x2_ref, hm_ref)
    z = x1_ref[...] + x2_ref[...]
    z = jnp.maximum(
        jnp.dot(z, l1w_ref[...], preferred_element_type=F32) + l1b_ref[...],
        0.0)
    out_ref[...] = (jnp.dot(z, l2w_ref[...], preferred_element_type=F32)
                    + l2b_ref[...])


def _tc_final(h, sc, mk, bcol, ohT, rs, re, x1, l1w, l1b, l2w, l2b):
    return pl.pallas_call(
        _final_body,
        in_specs=[
            pl.BlockSpec((NPAD, FD), None),
            pl.BlockSpec((NPAD, 1), None),
            pl.BlockSpec((NPAD, 1), None),
            pl.BlockSpec((NPAD, 1), None),
            pl.BlockSpec((NG, NPAD), None),
            pl.BlockSpec(memory_space=pltpu.SMEM),
            pl.BlockSpec(memory_space=pltpu.SMEM),
            pl.BlockSpec((NG, 2 * FD), None),
            pl.BlockSpec((2 * FD, FD), None),
            pl.BlockSpec((1, FD), None),
            pl.BlockSpec((FD, NG), None),
            pl.BlockSpec((1, NG), None),
        ],
        out_shape=jax.ShapeDtypeStruct((NG, NG), F32),
        scratch_shapes=[pltpu.VMEM((NPAD, FD), F32),
                        pltpu.VMEM((NG, 2 * FD), F32)],
    )(h, sc, mk, bcol, ohT, rs, re, x1, l1w, l1b, l2w, l2b)


def kernel(x, edge_index, edge_attr, batch, W1, b1, p1, W2, b2, p2,
           lin1_W, lin1_b, lin2_W, lin2_b):
    del edge_attr
    xpad = jnp.pad(x, ((0, NPAD - NN), (0, 0)))
    row = edge_index[0].astype(I32)
    col = edge_index[1].astype(I32)
    rowt = jnp.concatenate(
        [row, jnp.zeros((EPAD - EE,), I32)]).reshape(NCH, 128)
    rowt2 = jnp.stack([rowt, rowt + NPAD])
    colt = jnp.concatenate(
        [col, GARB + jnp.arange(EPAD - EE, dtype=I32) % (NPAD - NN)]
    ).reshape(NCH, 128)
    bpad = jnp.concatenate(
        [batch.astype(I32), jnp.full((NPAD - NN,), NG - 1, I32)])
    bcol = bpad.reshape(NPAD, 1)
    b2d = bpad.reshape(NI, 128)
    valid = (jnp.arange(NPAD) < NN).astype(F32)
    vcol = valid.reshape(NPAD, 1)
    v2d = valid.reshape(NI, 128)
    oh = (bpad[:, None] == jnp.arange(NG)[None, :]).astype(F32) \
        * valid[:, None]
    ohT = oh.T
    starts = jnp.searchsorted(bpad, jnp.arange(NG)).astype(I32)
    ends = jnp.searchsorted(bpad, jnp.arange(NG), side="right").astype(I32)
    blo = b2d[:, 0]
    bhi = b2d[:, -1]
    jlo = (starts[blo] // JT).astype(I32)
    jhi = ((ends[bhi] + JT - 1) // JT).astype(I32)
    rs = (starts // GT).astype(I32)
    re = ((ends + GT - 1) // GT).astype(I32)
    ones_col = jnp.ones((NPAD, 1), F32)
    b1r = b1.reshape(1, FD)
    p1r = p1.reshape(1, FD)
    b2r = b2.reshape(1, FD)
    p2r = p2.reshape(1, FD)

    degp1 = _sc_deg_call(colt)
    d10 = degp1[0].reshape(NPAD, 1)
    d11 = degp1[1].reshape(NPAD, 1)
    h1p, h1s, dis1 = _tc_mm(xpad, ones_col, ones_col, W1, d10, d11, vcol)
    idxt1 = jnp.stack(
        [rowt2, jnp.broadcast_to(colt[None], (NSC, NCH, 128))], axis=2)
    accp1 = _sc_msg_call(h1s.reshape(NSC * NPAD, HF), idxt1)
    h1, score1, srank1 = _tc_comb(accp1, h1p, dis1, vcol, b1r, p1r)
    rank1 = _tc_rank(srank1.reshape(NI, 128), b2d, srank1, bcol, jlo, jhi)
    mask1 = _tc_sel(rank1.reshape(NPAD, 1), vcol, oh, ohT)
    x1 = _tc_readout(h1, score1, mask1, bcol, ohT, rs, re)
    degp2, ceff = _sc_mask_call(mask1.reshape(NPAD), rowt, colt)
    d20 = degp2[0].reshape(NPAD, 1)
    d21 = degp2[1].reshape(NPAD, 1)
    h2p, h2s, dis2 = _tc_mm(h1, score1, mask1, W2, d20, d21, mask1)
    idxt2 = jnp.stack(
        [rowt2, jnp.broadcast_to(ceff[None], (NSC, NCH, 128))], axis=2)
    accp2 = _sc_msg_call(h2s.reshape(NSC * NPAD, HF), idxt2)
    h2, score2, srank2 = _tc_comb(accp2, h2p, dis2, mask1, b2r, p2r)
    m1_2d = mask1.reshape(NI, 128)
    rank2 = _tc_rank(srank2.reshape(NI, 128), b2d, srank2, bcol, jlo, jhi)
    mask2 = _tc_sel(rank2.reshape(NPAD, 1), mask1, oh, ohT)
    return _tc_final(h2, score2, mask2, bcol, ohT, rs, re, x1,
                     lin1_W, lin1_b.reshape(1, FD), lin2_W,
                     lin2_b.reshape(1, NG))
